# baseline (speedup 1.0000x reference)
import jax
import jax.numpy as jnp
from jax.experimental import pallas as pl

B, N, C, H, W = 4, 8, 384, 24, 24
BANK = 4
HW = H * W
BN = B * N


def _fused_kernel(value_ref, mask_ref, keys_ref, vals_ref, gate_ref, out_ref):
    v = value_ref[0]
    m = mask_ref[0]
    denom = jnp.sum(m)
    wsum = jnp.sum(v * m, axis=1, keepdims=True)
    fsum = jnp.sum(v, axis=1, keepdims=True)
    pooled = wsum / jnp.maximum(denom, 1e-6)
    fallback = fsum * (1.0 / HW)
    z = jnp.where(denom > 1e-5, pooled, fallback)
    zr = z.reshape(1, C)

    keys = keys_ref[0]
    vals = vals_ref[0]
    diff = zr - keys
    dist = jnp.sum(diff * diff, axis=1, keepdims=True)
    logits = -dist
    mx = jnp.max(logits, axis=0, keepdims=True)
    e = jnp.exp(logits - mx)
    wts = e / jnp.sum(e, axis=0, keepdims=True)
    readout = jnp.sum(wts * vals, axis=0, keepdims=True)
    out_ref[0] = zr + gate_ref[0, 0] * readout


def kernel(value_BNCHW, key_BCHW, pixfeat_BCHW, mask_BNHW, bank_keys, bank_vals, gate):
    del key_BCHW, pixfeat_BCHW
    value = value_BNCHW.reshape(BN, C, HW)
    mask = mask_BNHW.reshape(BN, 1, HW)
    keys = bank_keys.reshape(BN, BANK, C)
    vals = bank_vals.reshape(BN, BANK, C)
    gate2 = jnp.asarray(gate, jnp.float32).reshape(1, 1)

    out = pl.pallas_call(
        _fused_kernel,
        grid=(BN,),
        in_specs=[
            pl.BlockSpec((1, C, HW), lambda i: (i, 0, 0)),
            pl.BlockSpec((1, 1, HW), lambda i: (i, 0, 0)),
            pl.BlockSpec((1, BANK, C), lambda i: (i, 0, 0)),
            pl.BlockSpec((1, BANK, C), lambda i: (i, 0, 0)),
            pl.BlockSpec((1, 1), lambda i: (0, 0)),
        ],
        out_specs=pl.BlockSpec((1, 1, C), lambda i: (i, 0, 0)),
        out_shape=jax.ShapeDtypeStruct((BN, 1, C), jnp.float32),
    )(value, mask, keys, vals, gate2)
    return out.reshape(B, N, C)


# speedup vs baseline: 1.0068x; 1.0068x over previous
import jax
import jax.numpy as jnp
from jax.experimental import pallas as pl
from jax.experimental.pallas import tpu as pltpu

B, N, C, H, W = 4, 8, 384, 24, 24
BANK = 4
HW = H * W
BN = B * N


def _fused_kernel(value_ref, mask_ref, keys_ref, vals_ref, gate_ref, out_ref):
    v = value_ref[0]
    m = mask_ref[0]
    denom = jnp.sum(m)
    coef = jnp.where(denom > 1e-5, m / jnp.maximum(denom, 1e-6), 1.0 / HW)
    z = jnp.sum(v * coef, axis=1, keepdims=True)
    zr = z.reshape(1, C)

    keys = keys_ref[0]
    vals = vals_ref[0]
    diff = zr - keys
    dist = jnp.sum(diff * diff, axis=1, keepdims=True)
    logits = -dist
    mx = jnp.max(logits, axis=0, keepdims=True)
    e = jnp.exp(logits - mx)
    wts = e / jnp.sum(e, axis=0, keepdims=True)
    readout = jnp.sum(wts * vals, axis=0, keepdims=True)
    out_ref[0] = zr + gate_ref[0, 0] * readout


def kernel(value_BNCHW, key_BCHW, pixfeat_BCHW, mask_BNHW, bank_keys, bank_vals, gate):
    del key_BCHW, pixfeat_BCHW
    value = value_BNCHW.reshape(BN, C, HW)
    mask = mask_BNHW.reshape(BN, 1, HW)
    keys = bank_keys.reshape(BN, BANK, C)
    vals = bank_vals.reshape(BN, BANK, C)
    gate2 = jnp.asarray(gate, jnp.float32).reshape(1, 1)

    out = pl.pallas_call(
        _fused_kernel,
        grid=(BN,),
        in_specs=[
            pl.BlockSpec((1, C, HW), lambda i: (i, 0, 0)),
            pl.BlockSpec((1, 1, HW), lambda i: (i, 0, 0)),
            pl.BlockSpec((1, BANK, C), lambda i: (i, 0, 0)),
            pl.BlockSpec((1, BANK, C), lambda i: (i, 0, 0)),
            pl.BlockSpec((1, 1), lambda i: (0, 0)),
        ],
        out_specs=pl.BlockSpec((1, 1, C), lambda i: (i, 0, 0)),
        out_shape=jax.ShapeDtypeStruct((BN, 1, C), jnp.float32),
        compiler_params=pltpu.CompilerParams(
            dimension_semantics=("parallel",),
        ),
    )(value, mask, keys, vals, gate2)
    return out.reshape(B, N, C)


# speedup vs baseline: 1.3452x; 1.3361x over previous
import jax
import jax.numpy as jnp
from jax.experimental import pallas as pl
from jax.experimental.pallas import tpu as pltpu

B, N, C, H, W = 4, 8, 384, 24, 24
BANK = 4
HW = H * W
BN = B * N
BLK = 4


def _pool_kernel(value_ref, mask_ref, z_ref):
    v = value_ref[...]
    m = mask_ref[...]
    denom = jnp.sum(m, axis=2, keepdims=True)
    coef = jnp.where(denom > 1e-5, m / jnp.maximum(denom, 1e-6), 1.0 / HW)
    z_ref[...] = jnp.sum(v * coef, axis=2, keepdims=True)


def _retrieve_kernel(z_ref, keys_ref, vals_ref, gate_ref, out_ref):
    z = z_ref[...]
    keys = keys_ref[...]
    vals = vals_ref[...]
    diff = z - keys
    dist = jnp.sum(diff * diff, axis=2, keepdims=True)
    logits = -dist
    mx = jnp.max(logits, axis=1, keepdims=True)
    e = jnp.exp(logits - mx)
    wts = e / jnp.sum(e, axis=1, keepdims=True)
    readout = jnp.sum(wts * vals, axis=1, keepdims=True)
    out_ref[...] = z + gate_ref[0, 0] * readout


def kernel(value_BNCHW, key_BCHW, pixfeat_BCHW, mask_BNHW, bank_keys, bank_vals, gate):
    del key_BCHW, pixfeat_BCHW
    value = value_BNCHW.reshape(BN, C, HW)
    mask = mask_BNHW.reshape(BN, 1, HW)
    gate2 = jnp.asarray(gate, jnp.float32).reshape(1, 1)

    z_col = pl.pallas_call(
        _pool_kernel,
        grid=(BN // BLK,),
        in_specs=[
            pl.BlockSpec((BLK, C, HW), lambda i: (i, 0, 0)),
            pl.BlockSpec((BLK, 1, HW), lambda i: (i, 0, 0)),
        ],
        out_specs=pl.BlockSpec((BLK, C, 1), lambda i: (i, 0, 0)),
        out_shape=jax.ShapeDtypeStruct((BN, C, 1), jnp.float32),
        compiler_params=pltpu.CompilerParams(
            dimension_semantics=("parallel",),
        ),
    )(value, mask)

    z3 = z_col.reshape(BN, 1, C)
    out = pl.pallas_call(
        _retrieve_kernel,
        in_specs=[
            pl.BlockSpec((BN, 1, C), lambda: (0, 0, 0)),
            pl.BlockSpec((BN, BANK, C), lambda: (0, 0, 0)),
            pl.BlockSpec((BN, BANK, C), lambda: (0, 0, 0)),
            pl.BlockSpec((1, 1), lambda: (0, 0)),
        ],
        out_specs=pl.BlockSpec((BN, 1, C), lambda: (0, 0, 0)),
        out_shape=jax.ShapeDtypeStruct((BN, 1, C), jnp.float32),
    )(z3, bank_keys.reshape(BN, BANK, C), bank_vals.reshape(BN, BANK, C), gate2)
    return out.reshape(B, N, C)
